# baseline (speedup 1.0000x reference)
import functools

import jax
import jax.numpy as jnp
import numpy as np
from jax import lax
from jax.experimental import pallas as pl
from jax.experimental.pallas import tpu as pltpu
from jax.experimental.pallas import tpu_sc as plsc

KNN = 16
NPAD = 10240
INF = np.float32(np.inf)
IMAX = np.int32(2**31 - 1)
NORM17 = float(np.float32(np.float32(17.0) ** -0.5) * np.float32(np.float32(17.0) ** -0.5))


def _extract16(vals, idxs):
    bv, bi = [], []
    for _ in range(KNN):
        m = jnp.min(vals, axis=1)
        eq = vals == m[:, None]
        it = jnp.min(jnp.where(eq, idxs, INF), axis=1)
        vals = jnp.where(eq, INF, vals)
        bv.append(m)
        bi.append(it)
    return jnp.stack(bv, axis=1), jnp.stack(bi, axis=1)


def _knn_body(rows_ref, ct_ref, idx_ref, s_ref, *, rb, cb, npad):
    r = pl.program_id(1)
    rows2 = rows_ref[0] * 2.0
    growf = (r * rb + lax.broadcasted_iota(
        jnp.int32, (rb, cb), 0)).astype(jnp.float32)
    colpos = lax.broadcasted_iota(jnp.int32, (1, KNN), 1)
    best_v = jnp.full((rb, KNN), INF, jnp.float32)
    best_i = jnp.zeros((rb, KNN), jnp.float32)

    def chunk(c, carry):
        bv, bi = carry
        cs = ct_ref[0, :, pl.ds(c * cb, cb)]
        sqj = jnp.sum(cs * cs, axis=0)
        dot2 = lax.dot_general(rows2, cs, (((1,), (0,)), ((), ())),
                               preferred_element_type=jnp.float32)
        score = sqj[None, :] - dot2
        gcolf = (c * cb).astype(jnp.float32) + lax.broadcasted_iota(
            jnp.int32, (rb, cb), 1).astype(jnp.float32)
        score = jnp.where(gcolf == growf, INF, score)
        s_ref[...] = score
        m0 = jnp.min(score, axis=1)

        def w_cond(st):
            return st[3]

        def w_body(st):
            bv, bi, m, _ = st
            vals = s_ref[...]
            eq = vals == m[:, None]
            i = jnp.min(jnp.where(eq, gcolf, INF), axis=1)
            vals = jnp.where(eq, INF, vals)
            s_ref[...] = vals
            upd = (m < bv[:, KNN - 1])[:, None]
            bvr = pltpu.roll(bv, 1, 1)
            bir = pltpu.roll(bi, 1, 1)
            ge = bv >= m[:, None]
            prev_ge = (colpos > 0) & (bvr >= m[:, None])
            nbv = jnp.where(ge, jnp.where(prev_ge, bvr, m[:, None]), bv)
            nbi = jnp.where(ge, jnp.where(prev_ge, bir, i[:, None]), bi)
            bv = jnp.where(upd, nbv, bv)
            bi = jnp.where(upd, nbi, bi)
            m2 = jnp.min(vals, axis=1)
            cont = jnp.any(m2 < bv[:, KNN - 1])
            return bv, bi, m2, cont

        bv, bi, _, _ = lax.while_loop(
            w_cond, w_body, (bv, bi, m0, jnp.any(m0 < bv[:, KNN - 1])))
        return bv, bi

    best_v, best_i = lax.fori_loop(0, npad // cb, chunk, (best_v, best_i))
    idx_ref[0] = best_i.astype(jnp.int32)


def _knn(coords_pad, coords_padT, rb=128, cb=512):
    b, npad, _ = coords_pad.shape
    return pl.pallas_call(
        functools.partial(_knn_body, rb=rb, cb=cb, npad=npad),
        grid=(b, npad // rb),
        in_specs=[
            pl.BlockSpec((1, rb, 8), lambda i, r: (i, r, 0)),
            pl.BlockSpec((1, 8, npad), lambda i, r: (i, 0, 0)),
        ],
        out_specs=pl.BlockSpec((1, rb, KNN), lambda i, r: (i, r, 0)),
        out_shape=jax.ShapeDtypeStruct((b, npad, KNN), jnp.int32),
        scratch_shapes=[pltpu.VMEM((rb, cb), jnp.float32)],
    )(coords_pad, coords_padT)


def _linear_body(x_ref, w_ref, o_ref):
    o_ref[0] = jnp.dot(x_ref[0], w_ref[...], preferred_element_type=jnp.float32)


def _linear(x, wT, rb=1024):
    b, npad, f = x.shape
    d = wT.shape[1]
    return pl.pallas_call(
        _linear_body,
        grid=(b, npad // rb),
        in_specs=[
            pl.BlockSpec((1, rb, f), lambda i, r: (i, r, 0)),
            pl.BlockSpec((f, d), lambda i, r: (0, 0)),
        ],
        out_specs=pl.BlockSpec((1, rb, d), lambda i, r: (i, r, 0)),
        out_shape=jax.ShapeDtypeStruct((b, npad, d), jnp.float32),
    )(x, wT)


def _layer_body(agg_ref, xw_ref, bias_ref, w_ref, o_ref):
    h = jnp.maximum((agg_ref[0] + xw_ref[0]) * NORM17 + bias_ref[...], 0.0)
    o_ref[0] = jnp.dot(h, w_ref[...], preferred_element_type=jnp.float32)


def _layer(agg, xw, bias, wT, rb=1024):
    b, npad, f = xw.shape
    d = wT.shape[1]
    return pl.pallas_call(
        _layer_body,
        grid=(b, npad // rb),
        in_specs=[
            pl.BlockSpec((1, rb, f), lambda i, r: (i, r, 0)),
            pl.BlockSpec((1, rb, f), lambda i, r: (i, r, 0)),
            pl.BlockSpec((1, f), lambda i, r: (0, 0)),
            pl.BlockSpec((f, d), lambda i, r: (0, 0)),
        ],
        out_specs=pl.BlockSpec((1, rb, d), lambda i, r: (i, r, 0)),
        out_shape=jax.ShapeDtypeStruct((b, npad, d), jnp.float32),
    )(agg, xw, bias, wT)


def _final_body(agg_ref, xw_ref, bias_ref, w_ref, bf_ref, o_ref):
    h = jnp.maximum((agg_ref[0] + xw_ref[0]) * NORM17 + bias_ref[...], 0.0)
    o_ref[0] = (jnp.dot(h, w_ref[...], preferred_element_type=jnp.float32)
                + bf_ref[...])


def _final(agg, xw, bias, wT, bf, rb=1024):
    b, npad, f = xw.shape
    d = wT.shape[1]
    return pl.pallas_call(
        _final_body,
        grid=(b, npad // rb),
        in_specs=[
            pl.BlockSpec((1, rb, f), lambda i, r: (i, r, 0)),
            pl.BlockSpec((1, rb, f), lambda i, r: (i, r, 0)),
            pl.BlockSpec((1, f), lambda i, r: (0, 0)),
            pl.BlockSpec((f, d), lambda i, r: (0, 0)),
            pl.BlockSpec((1, d), lambda i, r: (0, 0)),
        ],
        out_specs=pl.BlockSpec((1, rb, d), lambda i, r: (i, r, 0)),
        out_shape=jax.ShapeDtypeStruct((b, npad, d), jnp.float32),
    )(agg, xw, bias, wT, bf)


def _sc_gather_sum(xw, idx3):
    npad, d = xw.shape
    info = plsc.get_sparse_core_info()
    nw = info.num_cores * info.num_subcores
    bpw = npad // nw
    mesh = plsc.VectorSubcoreMesh(core_axis_name="c", subcore_axis_name="s")

    @functools.partial(
        pl.kernel,
        out_type=jax.ShapeDtypeStruct((npad, d), jnp.float32),
        mesh=mesh,
        compiler_params=pltpu.CompilerParams(use_tc_tiling_on_sc=False),
        scratch_types=[
            pltpu.VMEM((KNN, bpw), jnp.int32),
            pltpu.VMEM((bpw, d), jnp.float32),
            pltpu.SemaphoreType.DMA,
        ],
    )
    def k(xw_hbm, idx_hbm, out_hbm, idx_v, acc_v, sem):
        wid = lax.axis_index("s") * info.num_cores + lax.axis_index("c")
        base = wid * bpw
        pltpu.sync_copy(idx_hbm.at[wid], idx_v)
        pltpu.async_copy(xw_hbm.at[idx_v.at[0]], acc_v, sem).wait()
        for j in range(1, KNN):
            pltpu.async_copy(xw_hbm.at[idx_v.at[j]], acc_v, sem, add=True).wait()
        pltpu.sync_copy(acc_v, out_hbm.at[pl.ds(base, bpw)])

    return k(xw, idx3)


def kernel(feats_batch, W1, b1, W2, b2, Wf, bf):
    b, n, f = feats_batch.shape
    fb = jnp.pad(feats_batch, ((0, 0), (0, NPAD - n), (0, 2)))
    valid = (jnp.arange(NPAD) < n)[None, :, None]
    coords = jnp.where(valid, fb[..., :3], 1e6)
    coords_pad = jnp.pad(coords, ((0, 0), (0, 0), (0, 5)))
    coords_padT = jnp.swapaxes(coords_pad, 1, 2)

    idx = _knn(coords_pad, coords_padT)
    nw = 32
    idxT = jnp.swapaxes(
        jnp.swapaxes(idx, 1, 2).reshape(b, KNN, nw, NPAD // nw), 1, 2)

    w1T = jnp.pad(W1, ((0, 64), (0, 2))).T
    xw1 = _linear(fb, w1T)
    agg1 = jnp.stack([_sc_gather_sum(xw1[i], idxT[i]) for i in range(b)])
    b1p = jnp.pad(b1, (0, 64))
    w2T = jnp.pad(W2.T, ((0, 64), (0, 0)))
    xw2 = _layer(agg1, xw1, b1p[None, :], w2T)
    agg2 = jnp.stack([_sc_gather_sum(xw2[i], idxT[i]) for i in range(b)])
    out = _final(agg2, xw2, b2[None, :], Wf.T, bf[None, :])
    return out[:, :n, :]


# speedup vs baseline: 2.5580x; 2.5580x over previous
import functools

import jax
import jax.numpy as jnp
import numpy as np
from jax import lax
from jax.experimental import pallas as pl
from jax.experimental.pallas import tpu as pltpu
from jax.experimental.pallas import tpu_sc as plsc

KNN = 16
NPAD = 10240
INF = np.float32(np.inf)
IMAX = np.int32(2**31 - 1)
NORM17 = float(np.float32(np.float32(17.0) ** -0.5) * np.float32(np.float32(17.0) ** -0.5))


def _extract16_lanes(vals, idxs):
    bv, bi = [], []
    for _ in range(KNN):
        m = jnp.min(vals, axis=1)
        eq = vals == m[:, None]
        it = jnp.min(jnp.where(eq, idxs, INF), axis=1)
        vals = jnp.where(eq, INF, vals)
        bv.append(m)
        bi.append(it)
    return jnp.stack(bv, axis=0), jnp.stack(bi, axis=0)


def _extract16_sub(vals, idxs):
    bv, bi = [], []
    for _ in range(KNN):
        m = jnp.min(vals, axis=0)
        eq = vals == m[None, :]
        it = jnp.min(jnp.where(eq, idxs, INF), axis=0)
        vals = jnp.where(eq, INF, vals)
        bv.append(m)
        bi.append(it)
    return jnp.stack(bv, axis=0), jnp.stack(bi, axis=0)


def _knn_body(rows_ref, ct_ref, idx_ref, *, rb, cb, npad):
    r = pl.program_id(1)
    rows2 = rows_ref[0] * 2.0
    growf = (r * rb + lax.broadcasted_iota(
        jnp.int32, (rb, cb), 0)).astype(jnp.float32)
    best_v = jnp.full((KNN, rb), INF, jnp.float32)
    best_i = jnp.zeros((KNN, rb), jnp.float32)

    def chunk(c, carry):
        bv, bi = carry
        cs = ct_ref[0, :, pl.ds(c * cb, cb)]
        sqj = jnp.sum(cs * cs, axis=0)
        dot2 = lax.dot_general(rows2, cs, (((1,), (0,)), ((), ())),
                               preferred_element_type=jnp.float32)
        score = sqj[None, :] - dot2
        gcolf = (c * cb).astype(jnp.float32) + lax.broadcasted_iota(
            jnp.int32, (rb, cb), 1).astype(jnp.float32)
        score = jnp.where(gcolf == growf, INF, score)
        cv, ci = _extract16_lanes(score, gcolf)
        return _extract16_sub(jnp.concatenate([bv, cv], axis=0),
                              jnp.concatenate([bi, ci], axis=0))

    best_v, best_i = lax.fori_loop(0, npad // cb, chunk, (best_v, best_i))
    idx_ref[0] = best_i.T.astype(jnp.int32)


def _knn(coords_pad, coords_padT, rb=128, cb=512):
    b, npad, _ = coords_pad.shape
    return pl.pallas_call(
        functools.partial(_knn_body, rb=rb, cb=cb, npad=npad),
        grid=(b, npad // rb),
        in_specs=[
            pl.BlockSpec((1, rb, 8), lambda i, r: (i, r, 0)),
            pl.BlockSpec((1, 8, npad), lambda i, r: (i, 0, 0)),
        ],
        out_specs=pl.BlockSpec((1, rb, KNN), lambda i, r: (i, r, 0)),
        out_shape=jax.ShapeDtypeStruct((b, npad, KNN), jnp.int32),
    )(coords_pad, coords_padT)


def _linear_body(x_ref, w_ref, o_ref):
    o_ref[0] = jnp.dot(x_ref[0], w_ref[...], preferred_element_type=jnp.float32)


def _linear(x, wT, rb=1024):
    b, npad, f = x.shape
    d = wT.shape[1]
    return pl.pallas_call(
        _linear_body,
        grid=(b, npad // rb),
        in_specs=[
            pl.BlockSpec((1, rb, f), lambda i, r: (i, r, 0)),
            pl.BlockSpec((f, d), lambda i, r: (0, 0)),
        ],
        out_specs=pl.BlockSpec((1, rb, d), lambda i, r: (i, r, 0)),
        out_shape=jax.ShapeDtypeStruct((b, npad, d), jnp.float32),
    )(x, wT)


def _layer_body(agg_ref, xw_ref, bias_ref, w_ref, o_ref):
    h = jnp.maximum((agg_ref[0] + xw_ref[0]) * NORM17 + bias_ref[...], 0.0)
    o_ref[0] = jnp.dot(h, w_ref[...], preferred_element_type=jnp.float32)


def _layer(agg, xw, bias, wT, rb=1024):
    b, npad, f = xw.shape
    d = wT.shape[1]
    return pl.pallas_call(
        _layer_body,
        grid=(b, npad // rb),
        in_specs=[
            pl.BlockSpec((1, rb, f), lambda i, r: (i, r, 0)),
            pl.BlockSpec((1, rb, f), lambda i, r: (i, r, 0)),
            pl.BlockSpec((1, f), lambda i, r: (0, 0)),
            pl.BlockSpec((f, d), lambda i, r: (0, 0)),
        ],
        out_specs=pl.BlockSpec((1, rb, d), lambda i, r: (i, r, 0)),
        out_shape=jax.ShapeDtypeStruct((b, npad, d), jnp.float32),
    )(agg, xw, bias, wT)


def _final_body(agg_ref, xw_ref, bias_ref, w_ref, bf_ref, o_ref):
    h = jnp.maximum((agg_ref[0] + xw_ref[0]) * NORM17 + bias_ref[...], 0.0)
    o_ref[0] = (jnp.dot(h, w_ref[...], preferred_element_type=jnp.float32)
                + bf_ref[...])


def _final(agg, xw, bias, wT, bf, rb=1024):
    b, npad, f = xw.shape
    d = wT.shape[1]
    return pl.pallas_call(
        _final_body,
        grid=(b, npad // rb),
        in_specs=[
            pl.BlockSpec((1, rb, f), lambda i, r: (i, r, 0)),
            pl.BlockSpec((1, rb, f), lambda i, r: (i, r, 0)),
            pl.BlockSpec((1, f), lambda i, r: (0, 0)),
            pl.BlockSpec((f, d), lambda i, r: (0, 0)),
            pl.BlockSpec((1, d), lambda i, r: (0, 0)),
        ],
        out_specs=pl.BlockSpec((1, rb, d), lambda i, r: (i, r, 0)),
        out_shape=jax.ShapeDtypeStruct((b, npad, d), jnp.float32),
    )(agg, xw, bias, wT, bf)


def _sc_gather_sum(xw, idx3):
    npad, d = xw.shape
    info = plsc.get_sparse_core_info()
    nw = info.num_cores * info.num_subcores
    bpw = npad // nw
    mesh = plsc.VectorSubcoreMesh(core_axis_name="c", subcore_axis_name="s")

    @functools.partial(
        pl.kernel,
        out_type=jax.ShapeDtypeStruct((npad, d), jnp.float32),
        mesh=mesh,
        compiler_params=pltpu.CompilerParams(use_tc_tiling_on_sc=False),
        scratch_types=[
            pltpu.VMEM((KNN, bpw), jnp.int32),
            pltpu.VMEM((bpw, d), jnp.float32),
            pltpu.SemaphoreType.DMA,
        ],
    )
    def k(xw_hbm, idx_hbm, out_hbm, idx_v, acc_v, sem):
        wid = lax.axis_index("s") * info.num_cores + lax.axis_index("c")
        base = wid * bpw
        pltpu.sync_copy(idx_hbm.at[wid], idx_v)
        pltpu.async_copy(xw_hbm.at[idx_v.at[0]], acc_v, sem).wait()
        for j in range(1, KNN):
            pltpu.async_copy(xw_hbm.at[idx_v.at[j]], acc_v, sem, add=True).wait()
        pltpu.sync_copy(acc_v, out_hbm.at[pl.ds(base, bpw)])

    return k(xw, idx3)


def kernel(feats_batch, W1, b1, W2, b2, Wf, bf):
    b, n, f = feats_batch.shape
    fb = jnp.pad(feats_batch, ((0, 0), (0, NPAD - n), (0, 2)))
    valid = (jnp.arange(NPAD) < n)[None, :, None]
    coords = jnp.where(valid, fb[..., :3], 1e6)
    coords_pad = jnp.pad(coords, ((0, 0), (0, 0), (0, 5)))
    coords_padT = jnp.swapaxes(coords_pad, 1, 2)

    idx = _knn(coords_pad, coords_padT)
    nw = 32
    idxT = jnp.swapaxes(
        jnp.swapaxes(idx, 1, 2).reshape(b, KNN, nw, NPAD // nw), 1, 2)

    w1T = jnp.pad(W1, ((0, 64), (0, 2))).T
    xw1 = _linear(fb, w1T)
    agg1 = jnp.stack([_sc_gather_sum(xw1[i], idxT[i]) for i in range(b)])
    b1p = jnp.pad(b1, (0, 64))
    w2T = jnp.pad(W2.T, ((0, 64), (0, 0)))
    xw2 = _layer(agg1, xw1, b1p[None, :], w2T)
    agg2 = jnp.stack([_sc_gather_sum(xw2[i], idxT[i]) for i in range(b)])
    out = _final(agg2, xw2, b2[None, :], Wf.T, bf[None, :])
    return out[:, :n, :]


# speedup vs baseline: 2.9342x; 1.1471x over previous
import functools

import jax
import jax.numpy as jnp
import numpy as np
from jax import lax
from jax.experimental import pallas as pl
from jax.experimental.pallas import tpu as pltpu
from jax.experimental.pallas import tpu_sc as plsc

KNN = 16
NPAD = 10240
INF = np.float32(np.inf)
IMAX = np.int32(2**31 - 1)
NORM17 = float(np.float32(np.float32(17.0) ** -0.5) * np.float32(np.float32(17.0) ** -0.5))


def _extract16_T(vals, idxs, ones_w):
    bv, bi = [], []
    for _ in range(KNN):
        m = jnp.min(vals, axis=0)
        eq = vals == m[None, :]
        egc = jnp.where(eq, idxs, 0.0)
        it = lax.dot_general(ones_w, egc, (((1,), (0,)), ((), ())),
                             preferred_element_type=jnp.float32)
        vals = jnp.where(eq, INF, vals)
        bv.append(m)
        bi.append(it[0])
    return jnp.stack(bv, axis=0), jnp.stack(bi, axis=0)


def _extract16_sub(vals, idxs):
    bv, bi = [], []
    for _ in range(KNN):
        m = jnp.min(vals, axis=0)
        eq = vals == m[None, :]
        it = jnp.min(jnp.where(eq, idxs, INF), axis=0)
        vals = jnp.where(eq, INF, vals)
        bv.append(m)
        bi.append(it)
    return jnp.stack(bv, axis=0), jnp.stack(bi, axis=0)


def _knn_body(rows_ref, ct_ref, idx_ref, *, rb, cb, npad):
    r = pl.program_id(1)
    rows2 = rows_ref[0] * 2.0
    growf = (r * rb + lax.broadcasted_iota(
        jnp.int32, (cb, rb), 1)).astype(jnp.float32)
    ones_w = jnp.ones((1, cb), jnp.float32)
    best_v = jnp.full((KNN, rb), INF, jnp.float32)
    best_i = jnp.zeros((KNN, rb), jnp.float32)

    def chunk(c, carry):
        bv, bi = carry
        cs = ct_ref[0, :, pl.ds(c * cb, cb)]
        sqj = jnp.sum(cs * cs, axis=0)
        dot2 = lax.dot_general(cs, rows2, (((0,), (1,)), ((), ())),
                               preferred_element_type=jnp.float32)
        score = sqj[:, None] - dot2
        gcolf = (c * cb).astype(jnp.float32) + lax.broadcasted_iota(
            jnp.int32, (cb, rb), 0).astype(jnp.float32)
        score = jnp.where(gcolf == growf, INF, score)
        cv, ci = _extract16_T(score, gcolf, ones_w)
        return _extract16_sub(jnp.concatenate([bv, cv], axis=0),
                              jnp.concatenate([bi, ci], axis=0))

    best_v, best_i = lax.fori_loop(0, npad // cb, chunk, (best_v, best_i))
    bi_c = jnp.minimum(jnp.maximum(best_i, 0.0), float(npad - 1))
    idx_ref[0] = bi_c.T.astype(jnp.int32)


def _knn(coords_pad, coords_padT, rb=128, cb=512):
    b, npad, _ = coords_pad.shape
    return pl.pallas_call(
        functools.partial(_knn_body, rb=rb, cb=cb, npad=npad),
        grid=(b, npad // rb),
        in_specs=[
            pl.BlockSpec((1, rb, 8), lambda i, r: (i, r, 0)),
            pl.BlockSpec((1, 8, npad), lambda i, r: (i, 0, 0)),
        ],
        out_specs=pl.BlockSpec((1, rb, KNN), lambda i, r: (i, r, 0)),
        out_shape=jax.ShapeDtypeStruct((b, npad, KNN), jnp.int32),
    )(coords_pad, coords_padT)


def _linear_body(x_ref, w_ref, o_ref):
    o_ref[0] = jnp.dot(x_ref[0], w_ref[...], preferred_element_type=jnp.float32)


def _linear(x, wT, rb=1024):
    b, npad, f = x.shape
    d = wT.shape[1]
    return pl.pallas_call(
        _linear_body,
        grid=(b, npad // rb),
        in_specs=[
            pl.BlockSpec((1, rb, f), lambda i, r: (i, r, 0)),
            pl.BlockSpec((f, d), lambda i, r: (0, 0)),
        ],
        out_specs=pl.BlockSpec((1, rb, d), lambda i, r: (i, r, 0)),
        out_shape=jax.ShapeDtypeStruct((b, npad, d), jnp.float32),
    )(x, wT)


def _layer_body(agg_ref, xw_ref, bias_ref, w_ref, o_ref):
    h = jnp.maximum((agg_ref[0] + xw_ref[0]) * NORM17 + bias_ref[...], 0.0)
    o_ref[0] = jnp.dot(h, w_ref[...], preferred_element_type=jnp.float32)


def _layer(agg, xw, bias, wT, rb=1024):
    b, npad, f = xw.shape
    d = wT.shape[1]
    return pl.pallas_call(
        _layer_body,
        grid=(b, npad // rb),
        in_specs=[
            pl.BlockSpec((1, rb, f), lambda i, r: (i, r, 0)),
            pl.BlockSpec((1, rb, f), lambda i, r: (i, r, 0)),
            pl.BlockSpec((1, f), lambda i, r: (0, 0)),
            pl.BlockSpec((f, d), lambda i, r: (0, 0)),
        ],
        out_specs=pl.BlockSpec((1, rb, d), lambda i, r: (i, r, 0)),
        out_shape=jax.ShapeDtypeStruct((b, npad, d), jnp.float32),
    )(agg, xw, bias, wT)


def _final_body(agg_ref, xw_ref, bias_ref, w_ref, bf_ref, o_ref):
    h = jnp.maximum((agg_ref[0] + xw_ref[0]) * NORM17 + bias_ref[...], 0.0)
    o_ref[0] = (jnp.dot(h, w_ref[...], preferred_element_type=jnp.float32)
                + bf_ref[...])


def _final(agg, xw, bias, wT, bf, rb=1024):
    b, npad, f = xw.shape
    d = wT.shape[1]
    return pl.pallas_call(
        _final_body,
        grid=(b, npad // rb),
        in_specs=[
            pl.BlockSpec((1, rb, f), lambda i, r: (i, r, 0)),
            pl.BlockSpec((1, rb, f), lambda i, r: (i, r, 0)),
            pl.BlockSpec((1, f), lambda i, r: (0, 0)),
            pl.BlockSpec((f, d), lambda i, r: (0, 0)),
            pl.BlockSpec((1, d), lambda i, r: (0, 0)),
        ],
        out_specs=pl.BlockSpec((1, rb, d), lambda i, r: (i, r, 0)),
        out_shape=jax.ShapeDtypeStruct((b, npad, d), jnp.float32),
    )(agg, xw, bias, wT, bf)


def _sc_gather_sum(xw, idx3):
    npad, d = xw.shape
    info = plsc.get_sparse_core_info()
    nw = info.num_cores * info.num_subcores
    bpw = npad // nw
    mesh = plsc.VectorSubcoreMesh(core_axis_name="c", subcore_axis_name="s")

    @functools.partial(
        pl.kernel,
        out_type=jax.ShapeDtypeStruct((npad, d), jnp.float32),
        mesh=mesh,
        compiler_params=pltpu.CompilerParams(use_tc_tiling_on_sc=False),
        scratch_types=[
            pltpu.VMEM((KNN, bpw), jnp.int32),
            pltpu.VMEM((bpw, d), jnp.float32),
            pltpu.SemaphoreType.DMA,
        ],
    )
    def k(xw_hbm, idx_hbm, out_hbm, idx_v, acc_v, sem):
        wid = lax.axis_index("s") * info.num_cores + lax.axis_index("c")
        base = wid * bpw
        pltpu.sync_copy(idx_hbm.at[wid], idx_v)
        pltpu.async_copy(xw_hbm.at[idx_v.at[0]], acc_v, sem).wait()
        for j in range(1, KNN):
            pltpu.async_copy(xw_hbm.at[idx_v.at[j]], acc_v, sem, add=True).wait()
        pltpu.sync_copy(acc_v, out_hbm.at[pl.ds(base, bpw)])

    return k(xw, idx3)


def kernel(feats_batch, W1, b1, W2, b2, Wf, bf):
    b, n, f = feats_batch.shape
    fb = jnp.pad(feats_batch, ((0, 0), (0, NPAD - n), (0, 2)))
    valid = (jnp.arange(NPAD) < n)[None, :, None]
    coords = jnp.where(valid, fb[..., :3], 1e6)
    coords_pad = jnp.pad(coords, ((0, 0), (0, 0), (0, 5)))
    coords_padT = jnp.swapaxes(coords_pad, 1, 2)

    idx = _knn(coords_pad, coords_padT)
    nw = 32
    idxT = jnp.swapaxes(
        jnp.swapaxes(idx, 1, 2).reshape(b, KNN, nw, NPAD // nw), 1, 2)

    w1T = jnp.pad(W1, ((0, 64), (0, 2))).T
    xw1 = _linear(fb, w1T)
    agg1 = jnp.stack([_sc_gather_sum(xw1[i], idxT[i]) for i in range(b)])
    b1p = jnp.pad(b1, (0, 64))
    w2T = jnp.pad(W2.T, ((0, 64), (0, 0)))
    xw2 = _layer(agg1, xw1, b1p[None, :], w2T)
    agg2 = jnp.stack([_sc_gather_sum(xw2[i], idxT[i]) for i in range(b)])
    out = _final(agg2, xw2, b2[None, :], Wf.T, bf[None, :])
    return out[:, :n, :]


# speedup vs baseline: 2.9745x; 1.0137x over previous
import functools

import jax
import jax.numpy as jnp
import numpy as np
from jax import lax
from jax.experimental import pallas as pl
from jax.experimental.pallas import tpu as pltpu
from jax.experimental.pallas import tpu_sc as plsc

KNN = 16
NPAD = 10240
INF = np.float32(np.inf)
IMAX = np.int32(2**31 - 1)
NORM17 = float(np.float32(np.float32(17.0) ** -0.5) * np.float32(np.float32(17.0) ** -0.5))


def _extract16_T(vals, idxs, ones_w):
    bv, bi = [], []
    for _ in range(KNN):
        m = jnp.min(vals, axis=0)
        eq = vals == m[None, :]
        it = jnp.min(jnp.where(eq, idxs, INF), axis=0)
        vals = jnp.where(eq, INF, vals)
        bv.append(m)
        bi.append(it)
    return jnp.stack(bv, axis=0), jnp.stack(bi, axis=0)


def _extract16_sub(vals, idxs):
    bv, bi = [], []
    for _ in range(KNN):
        m = jnp.min(vals, axis=0)
        eq = vals == m[None, :]
        it = jnp.min(jnp.where(eq, idxs, INF), axis=0)
        vals = jnp.where(eq, INF, vals)
        bv.append(m)
        bi.append(it)
    return jnp.stack(bv, axis=0), jnp.stack(bi, axis=0)


def _knn_body(rows_ref, ct_ref, idx_ref, *, rb, cb, npad):
    r = pl.program_id(1)
    rows2 = rows_ref[0] * 2.0
    growf = (r * rb + lax.broadcasted_iota(
        jnp.int32, (cb, rb), 1)).astype(jnp.float32)
    ones_w = jnp.ones((1, cb), jnp.float32)
    best_v = jnp.full((KNN, rb), INF, jnp.float32)
    best_i = jnp.zeros((KNN, rb), jnp.float32)

    def chunk(c, carry):
        bv, bi = carry
        cs = ct_ref[0, :, pl.ds(c * cb, cb)]
        sqj = jnp.sum(cs * cs, axis=0)
        dot2 = lax.dot_general(cs, rows2, (((0,), (1,)), ((), ())),
                               preferred_element_type=jnp.float32)
        score = sqj[:, None] - dot2
        gcolf = (c * cb).astype(jnp.float32) + lax.broadcasted_iota(
            jnp.int32, (cb, rb), 0).astype(jnp.float32)
        score = jnp.where(gcolf == growf, INF, score)
        cv, ci = _extract16_T(score, gcolf, ones_w)
        return _extract16_sub(jnp.concatenate([bv, cv], axis=0),
                              jnp.concatenate([bi, ci], axis=0))

    best_v, best_i = lax.fori_loop(0, npad // cb, chunk, (best_v, best_i))
    bi_c = jnp.minimum(jnp.maximum(best_i, 0.0), float(npad - 1))
    idx_ref[0] = bi_c.T.astype(jnp.int32)


def _knn(coords_pad, coords_padT, rb=128, cb=512):
    b, npad, _ = coords_pad.shape
    return pl.pallas_call(
        functools.partial(_knn_body, rb=rb, cb=cb, npad=npad),
        grid=(b, npad // rb),
        in_specs=[
            pl.BlockSpec((1, rb, 8), lambda i, r: (i, r, 0)),
            pl.BlockSpec((1, 8, npad), lambda i, r: (i, 0, 0)),
        ],
        out_specs=pl.BlockSpec((1, rb, KNN), lambda i, r: (i, r, 0)),
        out_shape=jax.ShapeDtypeStruct((b, npad, KNN), jnp.int32),
    )(coords_pad, coords_padT)


def _linear_body(x_ref, w_ref, o_ref):
    o_ref[0] = jnp.dot(x_ref[0], w_ref[...], preferred_element_type=jnp.float32)


def _linear(x, wT, rb=1024):
    b, npad, f = x.shape
    d = wT.shape[1]
    return pl.pallas_call(
        _linear_body,
        grid=(b, npad // rb),
        in_specs=[
            pl.BlockSpec((1, rb, f), lambda i, r: (i, r, 0)),
            pl.BlockSpec((f, d), lambda i, r: (0, 0)),
        ],
        out_specs=pl.BlockSpec((1, rb, d), lambda i, r: (i, r, 0)),
        out_shape=jax.ShapeDtypeStruct((b, npad, d), jnp.float32),
    )(x, wT)


def _layer_body(agg_ref, xw_ref, bias_ref, w_ref, o_ref):
    h = jnp.maximum((agg_ref[0] + xw_ref[0]) * NORM17 + bias_ref[...], 0.0)
    o_ref[0] = jnp.dot(h, w_ref[...], preferred_element_type=jnp.float32)


def _layer(agg, xw, bias, wT, rb=1024):
    b, npad, f = xw.shape
    d = wT.shape[1]
    return pl.pallas_call(
        _layer_body,
        grid=(b, npad // rb),
        in_specs=[
            pl.BlockSpec((1, rb, f), lambda i, r: (i, r, 0)),
            pl.BlockSpec((1, rb, f), lambda i, r: (i, r, 0)),
            pl.BlockSpec((1, f), lambda i, r: (0, 0)),
            pl.BlockSpec((f, d), lambda i, r: (0, 0)),
        ],
        out_specs=pl.BlockSpec((1, rb, d), lambda i, r: (i, r, 0)),
        out_shape=jax.ShapeDtypeStruct((b, npad, d), jnp.float32),
    )(agg, xw, bias, wT)


def _final_body(agg_ref, xw_ref, bias_ref, w_ref, bf_ref, o_ref):
    h = jnp.maximum((agg_ref[0] + xw_ref[0]) * NORM17 + bias_ref[...], 0.0)
    o_ref[0] = (jnp.dot(h, w_ref[...], preferred_element_type=jnp.float32)
                + bf_ref[...])


def _final(agg, xw, bias, wT, bf, rb=1024):
    b, npad, f = xw.shape
    d = wT.shape[1]
    return pl.pallas_call(
        _final_body,
        grid=(b, npad // rb),
        in_specs=[
            pl.BlockSpec((1, rb, f), lambda i, r: (i, r, 0)),
            pl.BlockSpec((1, rb, f), lambda i, r: (i, r, 0)),
            pl.BlockSpec((1, f), lambda i, r: (0, 0)),
            pl.BlockSpec((f, d), lambda i, r: (0, 0)),
            pl.BlockSpec((1, d), lambda i, r: (0, 0)),
        ],
        out_specs=pl.BlockSpec((1, rb, d), lambda i, r: (i, r, 0)),
        out_shape=jax.ShapeDtypeStruct((b, npad, d), jnp.float32),
    )(agg, xw, bias, wT, bf)


def _sc_gather_sum(xw, idx3):
    npad, d = xw.shape
    info = plsc.get_sparse_core_info()
    nw = info.num_cores * info.num_subcores
    bpw = npad // nw
    mesh = plsc.VectorSubcoreMesh(core_axis_name="c", subcore_axis_name="s")

    @functools.partial(
        pl.kernel,
        out_type=jax.ShapeDtypeStruct((npad, d), jnp.float32),
        mesh=mesh,
        compiler_params=pltpu.CompilerParams(use_tc_tiling_on_sc=False),
        scratch_types=[
            pltpu.VMEM((KNN, bpw), jnp.int32),
            pltpu.VMEM((bpw, d), jnp.float32),
            pltpu.SemaphoreType.DMA,
        ],
    )
    def k(xw_hbm, idx_hbm, out_hbm, idx_v, acc_v, sem):
        wid = lax.axis_index("s") * info.num_cores + lax.axis_index("c")
        base = wid * bpw
        pltpu.sync_copy(idx_hbm.at[wid], idx_v)
        pltpu.async_copy(xw_hbm.at[idx_v.at[0]], acc_v, sem).wait()
        for j in range(1, KNN):
            pltpu.async_copy(xw_hbm.at[idx_v.at[j]], acc_v, sem, add=True).wait()
        pltpu.sync_copy(acc_v, out_hbm.at[pl.ds(base, bpw)])

    return k(xw, idx3)


def kernel(feats_batch, W1, b1, W2, b2, Wf, bf):
    b, n, f = feats_batch.shape
    fb = jnp.pad(feats_batch, ((0, 0), (0, NPAD - n), (0, 2)))
    valid = (jnp.arange(NPAD) < n)[None, :, None]
    coords = jnp.where(valid, fb[..., :3], 1e6)
    coords_pad = jnp.pad(coords, ((0, 0), (0, 0), (0, 5)))
    coords_padT = jnp.swapaxes(coords_pad, 1, 2)

    idx = _knn(coords_pad, coords_padT)
    nw = 32
    idxT = jnp.swapaxes(
        jnp.swapaxes(idx, 1, 2).reshape(b, KNN, nw, NPAD // nw), 1, 2)

    w1T = jnp.pad(W1, ((0, 64), (0, 2))).T
    xw1 = _linear(fb, w1T)
    agg1 = jnp.stack([_sc_gather_sum(xw1[i], idxT[i]) for i in range(b)])
    b1p = jnp.pad(b1, (0, 64))
    w2T = jnp.pad(W2.T, ((0, 64), (0, 0)))
    xw2 = _layer(agg1, xw1, b1p[None, :], w2T)
    agg2 = jnp.stack([_sc_gather_sum(xw2[i], idxT[i]) for i in range(b)])
    out = _final(agg2, xw2, b2[None, :], Wf.T, bf[None, :])
    return out[:, :n, :]
